# baseline (speedup 1.0000x reference)
import functools

import jax
import jax.numpy as jnp
from jax import lax
from jax.experimental import pallas as pl
from jax.experimental.pallas import tpu as pltpu
from jax.experimental.pallas import tpu_sc as plsc

B, N, C, H, K = 1, 2048, 768, 12, 512
HD = C // H
RB = 512
PR = N // RB
SCALE = HD ** -0.5
F32 = jnp.float32


def _dot(a, b, prec=None):
    return lax.dot_general(a, b, (((1,), (0,)), ((), ())),
                           preferred_element_type=F32, precision=prec)


def _dot_nt(a, b, prec=None):
    return lax.dot_general(a, b, (((1,), (1,)), ((), ())),
                           preferred_element_type=F32, precision=prec)


def _argmin_t(t, embed, base):
    rown = _dot_nt(jnp.ones((1, HD), F32), t * t, lax.Precision.HIGHEST)
    norm2 = jnp.sum(embed * embed, axis=1, keepdims=True)
    score = rown - 2.0 * _dot_nt(embed, t) + norm2
    mind = jnp.min(score, axis=0, keepdims=True)
    kio = lax.broadcasted_iota(jnp.int32, score.shape, 0)
    idx = jnp.min(jnp.where(score == mind, kio, K), axis=0, keepdims=True)
    return idx + base


def _stage1(x_ref, w_ref, qe_ref, ke_ref,
            q_out, k_out, v_out, iq_out, ik_out):
    h = pl.program_id(1)
    x_blk = x_ref[...]
    q = _dot_nt(x_blk, w_ref[pl.ds(h * HD, HD), :])
    k = _dot_nt(x_blk, w_ref[pl.ds(C + h * HD, HD), :])
    v = _dot_nt(x_blk, w_ref[pl.ds(2 * C + h * HD, HD), :])
    q_out[0] = q
    k_out[0] = k
    v_out[0] = v
    iq_out[0] = _argmin_t(q, qe_ref[h], h * K)
    ik_out[0] = _argmin_t(k, ke_ref[h], (H + h) * K)


def _make_sc_gather():
    info = plsc.get_sparse_core_info()
    nw = info.num_cores * info.num_subcores
    rows_total = 2 * H * N
    gb = rows_total // nw
    cw = 128
    chunks = gb // cw
    mesh = plsc.VectorSubcoreMesh(core_axis_name="c", subcore_axis_name="s")

    @functools.partial(
        pl.kernel, mesh=mesh,
        compiler_params=pltpu.CompilerParams(use_tc_tiling_on_sc=False),
        out_type=jax.ShapeDtypeStruct((rows_total, HD), F32),
        scratch_types=[
            pltpu.VMEM((chunks, cw), jnp.int32),
            pltpu.VMEM((gb, HD), F32),
            pltpu.SemaphoreType.DMA,
        ],
    )
    def sc_gather(table_hbm, idx_hbm, out_hbm, idx_v, rows_v, sem):
        wid = lax.axis_index("s") * info.num_cores + lax.axis_index("c")
        pltpu.sync_copy(idx_hbm.at[wid], idx_v)
        copies = [
            pltpu.async_copy(table_hbm.at[idx_v.at[j]],
                             rows_v.at[pl.ds(j * cw, cw)], sem)
            for j in range(chunks)
        ]
        for c in copies:
            c.wait()
        pltpu.sync_copy(rows_v, out_hbm.at[pl.ds(wid * gb, gb)])

    return sc_gather, (nw, chunks, cw)


def _stage3(q_ref, k_ref, v_ref, qq_ref, qk_ref, wp_ref,
            out_ref, mseq_ref, msek_ref, kl_ref, e_s, catk_s):
    h = pl.program_id(0)
    t = pl.program_id(1)

    @pl.when((h == 0) & (t == 0))
    def _init():
        mseq_ref[0, 0] = 0.0
        msek_ref[0, 0] = 0.0
        kl_ref[0, 0] = 0.0

    rows = t * RB
    q = q_ref[0]
    qq = qq_ref[0]
    grow = rows + lax.broadcasted_iota(jnp.int32, (RB, 1), 0)
    keep = grow > 0
    cat_q = jnp.where(keep, qq, q)
    d = qq - q
    mseq_ref[0, 0] += jnp.sum(jnp.where(keep, d * d, 0.0))

    @pl.when(t == 0)
    def _per_head():
        k0 = k_ref[0]
        qk = qk_ref[0]
        rows_n = lax.broadcasted_iota(jnp.int32, (N, 1), 0)
        catk_s[...] = jnp.where(rows_n > 0, qk, k0)
        dk = qk - k0
        msek_ref[0, 0] += jnp.sum(jnp.where(rows_n > 0, dk * dk, 0.0))

    k = k_ref[0]
    v = v_ref[0]

    u = _dot_nt(q, k) * SCALE
    e = jnp.exp(u)
    s1 = jnp.sum(e, axis=1, keepdims=True)
    eu1 = jnp.sum(e * u, axis=1, keepdims=True)
    e_s[...] = e

    u2 = _dot_nt(cat_q, catk_s[...]) * SCALE
    e2 = jnp.exp(u2)
    s2 = jnp.sum(e2, axis=1, keepdims=True)
    p = jnp.sum(e_s[...] * e2, axis=1, keepdims=True)
    qa = e2 * (1.0 / s2)
    o = _dot(qa, v)

    kl_ref[0, 0] += jnp.sum(eu1 / s1 - jnp.log(s1) - p / (s1 * s2))

    o768 = _dot(o, wp_ref[...])

    @pl.when(h == 0)
    def _store():
        out_ref[pl.ds(rows, RB), :] = o768

    @pl.when(h > 0)
    def _accum():
        out_ref[pl.ds(rows, RB), :] += o768


def kernel(x, W_qkv, W_proj, b_proj, q_embed, k_embed):
    x2 = x.reshape(N, C)
    qkv_spec = [
        pl.BlockSpec((RB, C), lambda t, h: (t, 0)),
        pl.BlockSpec((3 * C, C), lambda t, h: (0, 0)),
        pl.BlockSpec((H, K, HD), lambda t, h: (0, 0, 0)),
        pl.BlockSpec((H, K, HD), lambda t, h: (0, 0, 0)),
    ]
    qall, kall, vall, iq, ik = pl.pallas_call(
        _stage1,
        grid=(PR, H),
        in_specs=qkv_spec,
        out_specs=[
            pl.BlockSpec((1, RB, HD), lambda t, h: (h, t, 0)),
            pl.BlockSpec((1, RB, HD), lambda t, h: (h, t, 0)),
            pl.BlockSpec((1, RB, HD), lambda t, h: (h, t, 0)),
            pl.BlockSpec((1, 1, RB), lambda t, h: (h, 0, t)),
            pl.BlockSpec((1, 1, RB), lambda t, h: (h, 0, t)),
        ],
        out_shape=[
            jax.ShapeDtypeStruct((H, N, HD), F32),
            jax.ShapeDtypeStruct((H, N, HD), F32),
            jax.ShapeDtypeStruct((H, N, HD), F32),
            jax.ShapeDtypeStruct((H, 1, N), jnp.int32),
            jax.ShapeDtypeStruct((H, 1, N), jnp.int32),
        ],
        compiler_params=pltpu.CompilerParams(
            dimension_semantics=("arbitrary", "arbitrary"),
        ),
    )(x2, W_qkv, q_embed, k_embed)

    table = jnp.concatenate([q_embed.reshape(H * K, HD),
                             k_embed.reshape(H * K, HD)])
    idx_flat = jnp.concatenate([iq.reshape(H * N), ik.reshape(H * N)])
    sc_gather, idx_shape = _make_sc_gather()
    quant = sc_gather(table, idx_flat.reshape(idx_shape))
    qq = quant[:H * N].reshape(H, N, HD)
    qk = quant[H * N:].reshape(H, N, HD)

    scalar_spec = pl.BlockSpec((1, 1), lambda h, t: (0, 0),
                               memory_space=pltpu.SMEM)
    out, mseq, msek, kl = pl.pallas_call(
        _stage3,
        grid=(H, PR),
        in_specs=[
            pl.BlockSpec((1, RB, HD), lambda h, t: (h, t, 0)),
            pl.BlockSpec((1, N, HD), lambda h, t: (h, 0, 0)),
            pl.BlockSpec((1, N, HD), lambda h, t: (h, 0, 0)),
            pl.BlockSpec((1, RB, HD), lambda h, t: (h, t, 0)),
            pl.BlockSpec((1, N, HD), lambda h, t: (h, 0, 0)),
            pl.BlockSpec((HD, C), lambda h, t: (h, 0)),
        ],
        out_specs=[
            pl.BlockSpec((N, C), lambda h, t: (0, 0)),
            scalar_spec, scalar_spec, scalar_spec,
        ],
        out_shape=[
            jax.ShapeDtypeStruct((N, C), F32),
            jax.ShapeDtypeStruct((1, 1), F32),
            jax.ShapeDtypeStruct((1, 1), F32),
            jax.ShapeDtypeStruct((1, 1), F32),
        ],
        scratch_shapes=[
            pltpu.VMEM((RB, N), F32),
            pltpu.VMEM((N, HD), F32),
        ],
        compiler_params=pltpu.CompilerParams(
            dimension_semantics=("arbitrary", "arbitrary"),
        ),
    )(qall, kall, vall, qq, qk, W_proj.T)

    M = N - 1
    quant_loss = (mseq[0, 0] + msek[0, 0]) / (H * M * HD) \
        + kl[0, 0] / (H * N * N)
    return (out + b_proj)[None], quant_loss


# speedup vs baseline: 1.0261x; 1.0261x over previous
import functools

import jax
import jax.numpy as jnp
from jax import lax
from jax.experimental import pallas as pl
from jax.experimental.pallas import tpu as pltpu
from jax.experimental.pallas import tpu_sc as plsc

B, N, C, H, K = 1, 2048, 768, 12, 512
HD = C // H
RB = 512
PR = N // RB
SCALE = HD ** -0.5
F32 = jnp.float32


def _dot(a, b, prec=None):
    return lax.dot_general(a, b, (((1,), (0,)), ((), ())),
                           preferred_element_type=F32, precision=prec)


def _dot_nt(a, b, prec=None):
    return lax.dot_general(a, b, (((1,), (1,)), ((), ())),
                           preferred_element_type=F32, precision=prec)


def _argmin_t(t, embed, base, keep_lane):
    rown = _dot_nt(jnp.ones((1, HD), F32), t * t, lax.Precision.HIGHEST)
    norm2 = jnp.sum(embed * embed, axis=1, keepdims=True)
    score = rown - 2.0 * _dot_nt(embed, t) + norm2
    mind = jnp.min(score, axis=0, keepdims=True)
    kio = lax.broadcasted_iota(jnp.int32, score.shape, 0)
    idx = jnp.min(jnp.where(score == mind, kio, K), axis=0, keepdims=True)
    mse = jnp.sum(jnp.where(keep_lane, mind, 0.0))
    return idx + base, mse


def _stage1(x_ref, w_ref, qe_ref, ke_ref,
            q_out, k_out, v_out, iq_out, ik_out, mseq_ref, msek_ref):
    t = pl.program_id(0)
    h = pl.program_id(1)

    @pl.when((t == 0) & (h == 0))
    def _init():
        mseq_ref[0, 0] = 0.0
        msek_ref[0, 0] = 0.0

    x_blk = x_ref[...]
    q = _dot_nt(x_blk, w_ref[pl.ds(h * HD, HD), :])
    k = _dot_nt(x_blk, w_ref[pl.ds(C + h * HD, HD), :])
    v = _dot_nt(x_blk, w_ref[pl.ds(2 * C + h * HD, HD), :])
    q_out[0] = q
    k_out[0] = k
    v_out[0] = v
    keep = (t * RB + lax.broadcasted_iota(jnp.int32, (1, RB), 1)) > 0
    iq, mq = _argmin_t(q, qe_ref[h], h * K, keep)
    ik, mk = _argmin_t(k, ke_ref[h], (H + h) * K, keep)
    iq_out[0] = iq
    ik_out[0] = ik
    mseq_ref[0, 0] += mq
    msek_ref[0, 0] += mk


def _make_sc_gather():
    info = plsc.get_sparse_core_info()
    nw = info.num_cores * info.num_subcores
    rows_total = 2 * H * N
    gb = rows_total // nw
    cw = 128
    chunks = gb // cw
    mesh = plsc.VectorSubcoreMesh(core_axis_name="c", subcore_axis_name="s")

    @functools.partial(
        pl.kernel, mesh=mesh,
        compiler_params=pltpu.CompilerParams(use_tc_tiling_on_sc=False),
        out_type=jax.ShapeDtypeStruct((rows_total, HD), F32),
        scratch_types=[
            pltpu.VMEM((chunks, cw), jnp.int32),
            pltpu.VMEM((gb, HD), F32),
            pltpu.SemaphoreType.DMA,
        ],
    )
    def sc_gather(table_hbm, idx_hbm, out_hbm, idx_v, rows_v, sem):
        wid = lax.axis_index("s") * info.num_cores + lax.axis_index("c")
        pltpu.sync_copy(idx_hbm.at[wid], idx_v)
        copies = [
            pltpu.async_copy(table_hbm.at[idx_v.at[j]],
                             rows_v.at[pl.ds(j * cw, cw)], sem)
            for j in range(chunks)
        ]
        for c in copies:
            c.wait()
        pltpu.sync_copy(rows_v, out_hbm.at[pl.ds(wid * gb, gb)])

    return sc_gather, (nw, chunks, cw)


def _stage3(q_ref, k_ref, v_ref, qq_ref, qk_ref, wp_ref,
            out_ref, kl_ref, e_s, catk_s):
    h = pl.program_id(0)
    t = pl.program_id(1)

    @pl.when((h == 0) & (t == 0))
    def _init():
        kl_ref[0, 0] = 0.0

    rows = t * RB
    q = q_ref[0]
    qq = qq_ref[0]
    grow = rows + lax.broadcasted_iota(jnp.int32, (RB, 1), 0)
    cat_q = jnp.where(grow > 0, qq, q)

    @pl.when(t == 0)
    def _per_head():
        rows_n = lax.broadcasted_iota(jnp.int32, (N, 1), 0)
        catk_s[...] = jnp.where(rows_n > 0, qk_ref[0], k_ref[0])

    k = k_ref[0]
    v = v_ref[0]

    u = _dot_nt(q, k) * SCALE
    e = jnp.exp(u)
    s1 = jnp.sum(e, axis=1, keepdims=True)
    eu1 = jnp.sum(e * u, axis=1, keepdims=True)
    e_s[...] = e

    u2 = _dot_nt(cat_q, catk_s[...]) * SCALE
    e2 = jnp.exp(u2)
    s2 = jnp.sum(e2, axis=1, keepdims=True)
    p = jnp.sum(e_s[...] * e2, axis=1, keepdims=True)
    qa = e2 * (1.0 / s2)
    o = _dot(qa, v)

    kl_ref[0, 0] += jnp.sum(eu1 / s1 - jnp.log(s1) - p / (s1 * s2))

    o768 = _dot(o, wp_ref[...])

    @pl.when(h == 0)
    def _store():
        out_ref[pl.ds(rows, RB), :] = o768

    @pl.when(h > 0)
    def _accum():
        out_ref[pl.ds(rows, RB), :] += o768


def kernel(x, W_qkv, W_proj, b_proj, q_embed, k_embed):
    x2 = x.reshape(N, C)
    qkv_spec = [
        pl.BlockSpec((RB, C), lambda t, h: (t, 0)),
        pl.BlockSpec((3 * C, C), lambda t, h: (0, 0)),
        pl.BlockSpec((H, K, HD), lambda t, h: (0, 0, 0)),
        pl.BlockSpec((H, K, HD), lambda t, h: (0, 0, 0)),
    ]
    s1_scalar = pl.BlockSpec((1, 1), lambda t, h: (0, 0),
                             memory_space=pltpu.SMEM)
    qall, kall, vall, iq, ik, mseq, msek = pl.pallas_call(
        _stage1,
        grid=(PR, H),
        in_specs=qkv_spec,
        out_specs=[
            pl.BlockSpec((1, RB, HD), lambda t, h: (h, t, 0)),
            pl.BlockSpec((1, RB, HD), lambda t, h: (h, t, 0)),
            pl.BlockSpec((1, RB, HD), lambda t, h: (h, t, 0)),
            pl.BlockSpec((1, 1, RB), lambda t, h: (h, 0, t)),
            pl.BlockSpec((1, 1, RB), lambda t, h: (h, 0, t)),
            s1_scalar, s1_scalar,
        ],
        out_shape=[
            jax.ShapeDtypeStruct((H, N, HD), F32),
            jax.ShapeDtypeStruct((H, N, HD), F32),
            jax.ShapeDtypeStruct((H, N, HD), F32),
            jax.ShapeDtypeStruct((H, 1, N), jnp.int32),
            jax.ShapeDtypeStruct((H, 1, N), jnp.int32),
            jax.ShapeDtypeStruct((1, 1), F32),
            jax.ShapeDtypeStruct((1, 1), F32),
        ],
        compiler_params=pltpu.CompilerParams(
            dimension_semantics=("arbitrary", "arbitrary"),
        ),
    )(x2, W_qkv, q_embed, k_embed)

    table = jnp.concatenate([q_embed.reshape(H * K, HD),
                             k_embed.reshape(H * K, HD)])
    idx_flat = jnp.concatenate([iq.reshape(H * N), ik.reshape(H * N)])
    sc_gather, idx_shape = _make_sc_gather()
    quant = sc_gather(table, idx_flat.reshape(idx_shape))
    qq = quant[:H * N].reshape(H, N, HD)
    qk = quant[H * N:].reshape(H, N, HD)

    scalar_spec = pl.BlockSpec((1, 1), lambda h, t: (0, 0),
                               memory_space=pltpu.SMEM)
    out, kl = pl.pallas_call(
        _stage3,
        grid=(H, PR),
        in_specs=[
            pl.BlockSpec((1, RB, HD), lambda h, t: (h, t, 0)),
            pl.BlockSpec((1, N, HD), lambda h, t: (h, 0, 0)),
            pl.BlockSpec((1, N, HD), lambda h, t: (h, 0, 0)),
            pl.BlockSpec((1, RB, HD), lambda h, t: (h, t, 0)),
            pl.BlockSpec((1, N, HD), lambda h, t: (h, 0, 0)),
            pl.BlockSpec((HD, C), lambda h, t: (h, 0)),
        ],
        out_specs=[
            pl.BlockSpec((N, C), lambda h, t: (0, 0)),
            scalar_spec,
        ],
        out_shape=[
            jax.ShapeDtypeStruct((N, C), F32),
            jax.ShapeDtypeStruct((1, 1), F32),
        ],
        scratch_shapes=[
            pltpu.VMEM((RB, N), F32),
            pltpu.VMEM((N, HD), F32),
        ],
        compiler_params=pltpu.CompilerParams(
            dimension_semantics=("arbitrary", "arbitrary"),
        ),
    )(qall, kall, vall, qq, qk, W_proj.T)

    M = N - 1
    quant_loss = (mseq[0, 0] + msek[0, 0]) / (H * M * HD) \
        + kl[0, 0] / (H * N * N)
    return (out + b_proj)[None], quant_loss


# speedup vs baseline: 1.1056x; 1.0774x over previous
import functools

import jax
import jax.numpy as jnp
from jax import lax
from jax.experimental import pallas as pl
from jax.experimental.pallas import tpu as pltpu
from jax.experimental.pallas import tpu_sc as plsc

B, N, C, H, K = 1, 2048, 768, 12, 512
HD = C // H
RB = 512
PR = N // RB
SCALE = HD ** -0.5
F32 = jnp.float32


def _dot(a, b, prec=None):
    return lax.dot_general(a, b, (((1,), (0,)), ((), ())),
                           preferred_element_type=F32, precision=prec)


def _dot_nt(a, b, prec=None):
    return lax.dot_general(a, b, (((1,), (1,)), ((), ())),
                           preferred_element_type=F32, precision=prec)


def _argmin_t(t, embed, base, keep_lane):
    rown = _dot_nt(jnp.ones((1, HD), F32), t * t, lax.Precision.HIGHEST)
    norm2 = jnp.sum(embed * embed, axis=1, keepdims=True)
    score = rown - 2.0 * _dot_nt(embed, t) + norm2
    mind = jnp.min(score, axis=0, keepdims=True)
    kio = lax.broadcasted_iota(jnp.int32, score.shape, 0)
    idx = jnp.min(jnp.where(score == mind, kio, K), axis=0, keepdims=True)
    mse = jnp.sum(jnp.where(keep_lane, mind, 0.0))
    return idx + base, mse


def _stage1(x_ref, w_ref, qe_ref, ke_ref,
            q_out, k_out, v_out, iq_out, ik_out, mseq_ref, msek_ref):
    t = pl.program_id(0)
    h = pl.program_id(1)

    @pl.when((t == 0) & (h == 0))
    def _init():
        mseq_ref[0, 0] = 0.0
        msek_ref[0, 0] = 0.0

    x_blk = x_ref[...]
    q = _dot_nt(x_blk, w_ref[pl.ds(h * HD, HD), :])
    k = _dot_nt(x_blk, w_ref[pl.ds(C + h * HD, HD), :])
    v = _dot_nt(x_blk, w_ref[pl.ds(2 * C + h * HD, HD), :])
    q_out[0] = q
    k_out[0] = k
    v_out[0] = v
    keep = (t * RB + lax.broadcasted_iota(jnp.int32, (1, RB), 1)) > 0
    iq, mq = _argmin_t(q, qe_ref[h], h * K, keep)
    ik, mk = _argmin_t(k, ke_ref[h], (H + h) * K, keep)
    iq_out[0] = iq
    ik_out[0] = ik
    mseq_ref[0, 0] += mq
    msek_ref[0, 0] += mk


def _make_sc_gather():
    info = plsc.get_sparse_core_info()
    nw = info.num_cores * info.num_subcores
    rows_total = 2 * H * N
    gb = rows_total // nw
    cw = 128
    chunks = gb // cw
    mesh = plsc.VectorSubcoreMesh(core_axis_name="c", subcore_axis_name="s")

    @functools.partial(
        pl.kernel, mesh=mesh,
        compiler_params=pltpu.CompilerParams(use_tc_tiling_on_sc=False),
        out_type=jax.ShapeDtypeStruct((rows_total, HD), F32),
        scratch_types=[
            pltpu.VMEM((chunks, cw), jnp.int32),
            pltpu.VMEM((gb, HD), F32),
            pltpu.SemaphoreType.DMA,
        ],
    )
    def sc_gather(table_hbm, idx_hbm, out_hbm, idx_v, rows_v, sem):
        wid = lax.axis_index("s") * info.num_cores + lax.axis_index("c")
        pltpu.sync_copy(idx_hbm.at[wid], idx_v)
        copies = [
            pltpu.async_copy(table_hbm.at[idx_v.at[j]],
                             rows_v.at[pl.ds(j * cw, cw)], sem)
            for j in range(chunks)
        ]
        for c in copies:
            c.wait()
        pltpu.sync_copy(rows_v, out_hbm.at[pl.ds(wid * gb, gb)])

    return sc_gather, (nw, chunks, cw)


def _stage3(q_ref, k_ref, v_ref, qq_ref, qk_ref, wp_ref,
            out_ref, kl_ref, e_s, catk_s):
    h = pl.program_id(0)
    t = pl.program_id(1)

    @pl.when((h == 0) & (t == 0))
    def _init():
        kl_ref[0, 0] = 0.0

    rows = t * RB
    q = q_ref[0]
    qq = qq_ref[0]
    grow = rows + lax.broadcasted_iota(jnp.int32, (RB, 1), 0)
    cat_q = jnp.where(grow > 0, qq, q)

    @pl.when(t == 0)
    def _per_head():
        rows_n = lax.broadcasted_iota(jnp.int32, (N, 1), 0)
        catk_s[...] = jnp.where(rows_n > 0, qk_ref[0], k_ref[0])

    k = k_ref[0]
    v = v_ref[0]

    u = _dot_nt(q, k) * SCALE
    e = jnp.exp(u)
    s1 = jnp.sum(e, axis=1, keepdims=True)
    eu1 = jnp.sum(e * u, axis=1, keepdims=True)
    e_s[...] = e

    u2 = _dot_nt(cat_q, catk_s[...]) * SCALE
    e2 = jnp.exp(u2)
    s2 = jnp.sum(e2, axis=1, keepdims=True)
    p = jnp.sum(e_s[...] * e2, axis=1, keepdims=True)
    o = _dot(e2, v) * (1.0 / s2)

    kl_ref[0, 0] += jnp.sum(eu1 / s1 - jnp.log(s1) - p / (s1 * s2))

    o768 = _dot(o, wp_ref[...])

    @pl.when(h == 0)
    def _store():
        out_ref[pl.ds(rows, RB), :] = o768

    @pl.when(h > 0)
    def _accum():
        out_ref[pl.ds(rows, RB), :] += o768


def kernel(x, W_qkv, W_proj, b_proj, q_embed, k_embed):
    x2 = x.reshape(N, C)
    qkv_spec = [
        pl.BlockSpec((RB, C), lambda t, h: (t, 0)),
        pl.BlockSpec((3 * C, C), lambda t, h: (0, 0)),
        pl.BlockSpec((H, K, HD), lambda t, h: (0, 0, 0)),
        pl.BlockSpec((H, K, HD), lambda t, h: (0, 0, 0)),
    ]
    s1_scalar = pl.BlockSpec((1, 1), lambda t, h: (0, 0),
                             memory_space=pltpu.SMEM)
    qall, kall, vall, iq, ik, mseq, msek = pl.pallas_call(
        _stage1,
        grid=(PR, H),
        in_specs=qkv_spec,
        out_specs=[
            pl.BlockSpec((1, RB, HD), lambda t, h: (h, t, 0)),
            pl.BlockSpec((1, RB, HD), lambda t, h: (h, t, 0)),
            pl.BlockSpec((1, RB, HD), lambda t, h: (h, t, 0)),
            pl.BlockSpec((1, 1, RB), lambda t, h: (h, 0, t)),
            pl.BlockSpec((1, 1, RB), lambda t, h: (h, 0, t)),
            s1_scalar, s1_scalar,
        ],
        out_shape=[
            jax.ShapeDtypeStruct((H, N, HD), F32),
            jax.ShapeDtypeStruct((H, N, HD), F32),
            jax.ShapeDtypeStruct((H, N, HD), F32),
            jax.ShapeDtypeStruct((H, 1, N), jnp.int32),
            jax.ShapeDtypeStruct((H, 1, N), jnp.int32),
            jax.ShapeDtypeStruct((1, 1), F32),
            jax.ShapeDtypeStruct((1, 1), F32),
        ],
        compiler_params=pltpu.CompilerParams(
            dimension_semantics=("arbitrary", "arbitrary"),
        ),
    )(x2, W_qkv, q_embed, k_embed)

    table = jnp.concatenate([q_embed.reshape(H * K, HD),
                             k_embed.reshape(H * K, HD)])
    idx_flat = jnp.concatenate([iq.reshape(H * N), ik.reshape(H * N)])
    sc_gather, idx_shape = _make_sc_gather()
    quant = sc_gather(table, idx_flat.reshape(idx_shape))
    qq = quant[:H * N].reshape(H, N, HD)
    qk = quant[H * N:].reshape(H, N, HD)

    scalar_spec = pl.BlockSpec((1, 1), lambda h, t: (0, 0),
                               memory_space=pltpu.SMEM)
    out, kl = pl.pallas_call(
        _stage3,
        grid=(H, PR),
        in_specs=[
            pl.BlockSpec((1, RB, HD), lambda h, t: (h, t, 0)),
            pl.BlockSpec((1, N, HD), lambda h, t: (h, 0, 0)),
            pl.BlockSpec((1, N, HD), lambda h, t: (h, 0, 0)),
            pl.BlockSpec((1, RB, HD), lambda h, t: (h, t, 0)),
            pl.BlockSpec((1, N, HD), lambda h, t: (h, 0, 0)),
            pl.BlockSpec((HD, C), lambda h, t: (h, 0)),
        ],
        out_specs=[
            pl.BlockSpec((N, C), lambda h, t: (0, 0)),
            scalar_spec,
        ],
        out_shape=[
            jax.ShapeDtypeStruct((N, C), F32),
            jax.ShapeDtypeStruct((1, 1), F32),
        ],
        scratch_shapes=[
            pltpu.VMEM((RB, N), F32),
            pltpu.VMEM((N, HD), F32),
        ],
        compiler_params=pltpu.CompilerParams(
            dimension_semantics=("arbitrary", "arbitrary"),
        ),
    )(qall, kall, vall, qq, qk, W_proj.T)

    M = N - 1
    quant_loss = (mseq[0, 0] + msek[0, 0]) / (H * M * HD) \
        + kl[0, 0] / (H * N * N)
    return (out + b_proj)[None], quant_loss


# speedup vs baseline: 1.1287x; 1.0209x over previous
import functools

import jax
import jax.numpy as jnp
from jax import lax
from jax.experimental import pallas as pl
from jax.experimental.pallas import tpu as pltpu
from jax.experimental.pallas import tpu_sc as plsc

B, N, C, H, K = 1, 2048, 768, 12, 512
HD = C // H
RB = 512
PR = N // RB
SCALE = HD ** -0.5
F32 = jnp.float32


def _dot(a, b, prec=None):
    return lax.dot_general(a, b, (((1,), (0,)), ((), ())),
                           preferred_element_type=F32, precision=prec)


def _dot_nt(a, b, prec=None):
    return lax.dot_general(a, b, (((1,), (1,)), ((), ())),
                           preferred_element_type=F32, precision=prec)


def _argmin_t(t, embed, base, keep_lane):
    rown = _dot_nt(jnp.ones((1, HD), F32), t * t, lax.Precision.HIGHEST)
    norm2 = jnp.sum(embed * embed, axis=1, keepdims=True)
    score = rown - 2.0 * _dot_nt(embed, t) + norm2
    mind = jnp.min(score, axis=0, keepdims=True)
    kio = lax.broadcasted_iota(jnp.int32, score.shape, 0)
    idx = jnp.min(jnp.where(score == mind, kio, K), axis=0, keepdims=True)
    mse = jnp.sum(jnp.where(keep_lane, mind, 0.0))
    return idx + base, mse


def _stage1(x_ref, w_ref, qe_ref, ke_ref,
            q_out, k_out, v_out, iq_out, ik_out, mseq_ref, msek_ref):
    t = pl.program_id(0)
    h = pl.program_id(1)

    @pl.when((t == 0) & (h == 0))
    def _init():
        mseq_ref[0, 0] = 0.0
        msek_ref[0, 0] = 0.0

    x_blk = x_ref[...]
    q = _dot_nt(x_blk, w_ref[pl.ds(h * HD, HD), :])
    k = _dot_nt(x_blk, w_ref[pl.ds(C + h * HD, HD), :])
    v = _dot_nt(x_blk, w_ref[pl.ds(2 * C + h * HD, HD), :])
    q_out[0] = q
    k_out[0] = k
    v_out[0] = v
    keep = (t * RB + lax.broadcasted_iota(jnp.int32, (1, RB), 1)) > 0
    iq, mq = _argmin_t(q, qe_ref[h], h * K, keep)
    ik, mk = _argmin_t(k, ke_ref[h], (H + h) * K, keep)
    iq_out[0] = iq
    ik_out[0] = ik
    mseq_ref[0, 0] += mq
    msek_ref[0, 0] += mk


def _make_sc_gather():
    info = plsc.get_sparse_core_info()
    nw = info.num_cores * info.num_subcores
    rows_total = 2 * H * N
    gb = rows_total // nw
    cw = 128
    chunks = gb // cw
    mesh = plsc.VectorSubcoreMesh(core_axis_name="c", subcore_axis_name="s")

    @functools.partial(
        pl.kernel, mesh=mesh,
        compiler_params=pltpu.CompilerParams(use_tc_tiling_on_sc=False),
        out_type=jax.ShapeDtypeStruct((rows_total, HD), F32),
        scratch_types=[
            pltpu.VMEM((chunks, cw), jnp.int32),
            pltpu.VMEM((gb, HD), F32),
            pltpu.SemaphoreType.DMA,
        ],
    )
    def sc_gather(table_hbm, idx_hbm, out_hbm, idx_v, rows_v, sem):
        wid = lax.axis_index("s") * info.num_cores + lax.axis_index("c")
        pltpu.sync_copy(idx_hbm.at[wid], idx_v)
        copies = [
            pltpu.async_copy(table_hbm.at[idx_v.at[j]],
                             rows_v.at[pl.ds(j * cw, cw)], sem)
            for j in range(chunks)
        ]
        for c in copies:
            c.wait()
        pltpu.sync_copy(rows_v, out_hbm.at[pl.ds(wid * gb, gb)])

    return sc_gather, (nw, chunks, cw)


def _stage3(q_ref, k_ref, v_ref, qq_ref, qk_ref, wp_ref,
            out_ref, kl_ref, e_s, catk_s):
    h = pl.program_id(0)
    t = pl.program_id(1)

    @pl.when((h == 0) & (t == 0))
    def _init():
        kl_ref[0, 0] = 0.0

    rows = t * RB
    q = q_ref[0]
    qq = qq_ref[0]
    grow = rows + lax.broadcasted_iota(jnp.int32, (RB, 1), 0)
    cat_q = jnp.where(grow > 0, qq, q)

    @pl.when(t == 0)
    def _per_head():
        rows_n = lax.broadcasted_iota(jnp.int32, (N, 1), 0)
        catk_s[...] = jnp.where(rows_n > 0, qk_ref[0], k_ref[0])

    k = k_ref[0]
    v = v_ref[0]

    u = _dot_nt(q * SCALE, k)
    e = jnp.exp(u)
    s1 = jnp.sum(e, axis=1, keepdims=True)
    eu1 = jnp.sum(e * u, axis=1, keepdims=True)
    e_s[...] = e

    u2 = _dot_nt(cat_q * SCALE, catk_s[...])
    e2 = jnp.exp(u2)
    s2 = jnp.sum(e2, axis=1, keepdims=True)
    p = jnp.sum(e_s[...] * e2, axis=1, keepdims=True)
    o = _dot(e2, v) * (1.0 / s2)

    kl_ref[0, 0] += jnp.sum(eu1 / s1 - jnp.log(s1) - p / (s1 * s2))

    o768 = _dot(o, wp_ref[...])

    @pl.when(h == 0)
    def _store():
        out_ref[pl.ds(rows, RB), :] = o768

    @pl.when(h > 0)
    def _accum():
        out_ref[pl.ds(rows, RB), :] += o768


def kernel(x, W_qkv, W_proj, b_proj, q_embed, k_embed):
    x2 = x.reshape(N, C)
    qkv_spec = [
        pl.BlockSpec((RB, C), lambda t, h: (t, 0)),
        pl.BlockSpec((3 * C, C), lambda t, h: (0, 0)),
        pl.BlockSpec((H, K, HD), lambda t, h: (0, 0, 0)),
        pl.BlockSpec((H, K, HD), lambda t, h: (0, 0, 0)),
    ]
    s1_scalar = pl.BlockSpec((1, 1), lambda t, h: (0, 0),
                             memory_space=pltpu.SMEM)
    qall, kall, vall, iq, ik, mseq, msek = pl.pallas_call(
        _stage1,
        grid=(PR, H),
        in_specs=qkv_spec,
        out_specs=[
            pl.BlockSpec((1, RB, HD), lambda t, h: (h, t, 0)),
            pl.BlockSpec((1, RB, HD), lambda t, h: (h, t, 0)),
            pl.BlockSpec((1, RB, HD), lambda t, h: (h, t, 0)),
            pl.BlockSpec((1, 1, RB), lambda t, h: (h, 0, t)),
            pl.BlockSpec((1, 1, RB), lambda t, h: (h, 0, t)),
            s1_scalar, s1_scalar,
        ],
        out_shape=[
            jax.ShapeDtypeStruct((H, N, HD), F32),
            jax.ShapeDtypeStruct((H, N, HD), F32),
            jax.ShapeDtypeStruct((H, N, HD), F32),
            jax.ShapeDtypeStruct((H, 1, N), jnp.int32),
            jax.ShapeDtypeStruct((H, 1, N), jnp.int32),
            jax.ShapeDtypeStruct((1, 1), F32),
            jax.ShapeDtypeStruct((1, 1), F32),
        ],
        compiler_params=pltpu.CompilerParams(
            dimension_semantics=("arbitrary", "arbitrary"),
        ),
    )(x2, W_qkv, q_embed, k_embed)

    table = jnp.concatenate([q_embed.reshape(H * K, HD),
                             k_embed.reshape(H * K, HD)])
    idx_flat = jnp.concatenate([iq.reshape(H * N), ik.reshape(H * N)])
    sc_gather, idx_shape = _make_sc_gather()
    quant = sc_gather(table, idx_flat.reshape(idx_shape))
    qq = quant[:H * N].reshape(H, N, HD)
    qk = quant[H * N:].reshape(H, N, HD)

    scalar_spec = pl.BlockSpec((1, 1), lambda h, t: (0, 0),
                               memory_space=pltpu.SMEM)
    out, kl = pl.pallas_call(
        _stage3,
        grid=(H, PR),
        in_specs=[
            pl.BlockSpec((1, RB, HD), lambda h, t: (h, t, 0)),
            pl.BlockSpec((1, N, HD), lambda h, t: (h, 0, 0)),
            pl.BlockSpec((1, N, HD), lambda h, t: (h, 0, 0)),
            pl.BlockSpec((1, RB, HD), lambda h, t: (h, t, 0)),
            pl.BlockSpec((1, N, HD), lambda h, t: (h, 0, 0)),
            pl.BlockSpec((HD, C), lambda h, t: (h, 0)),
        ],
        out_specs=[
            pl.BlockSpec((N, C), lambda h, t: (0, 0)),
            scalar_spec,
        ],
        out_shape=[
            jax.ShapeDtypeStruct((N, C), F32),
            jax.ShapeDtypeStruct((1, 1), F32),
        ],
        scratch_shapes=[
            pltpu.VMEM((RB, N), F32),
            pltpu.VMEM((N, HD), F32),
        ],
        compiler_params=pltpu.CompilerParams(
            dimension_semantics=("arbitrary", "arbitrary"),
        ),
    )(qall, kall, vall, qq, qk, W_proj.T)

    M = N - 1
    quant_loss = (mseq[0, 0] + msek[0, 0]) / (H * M * HD) \
        + kl[0, 0] / (H * N * N)
    return (out + b_proj)[None], quant_loss
